# baseline (speedup 1.0000x reference)
import functools

import jax
import jax.numpy as jnp
from jax import lax
from jax.experimental import pallas as pl
from jax.experimental.pallas import tpu as pltpu
from jax.experimental.pallas import tpu_sc as plsc

_NUNIQUE = 1000
_BATCH = 16384
_NW = 32
_COLS_PER_W = _BATCH // _NW
_CHUNK = 128
_NCHUNK = _COLS_PER_W // _CHUNK
_L = 16
_BANDS = (256, 256, 248, 240)
_NB = len(_BANDS)
_LOS = tuple(sum(_BANDS[:i]) for i in range(_NB))

_mesh = plsc.VectorSubcoreMesh(core_axis_name="c", subcore_axis_name="s")


@functools.partial(
    pl.kernel,
    mesh=_mesh,
    compiler_params=pltpu.CompilerParams(
        needs_layout_passes=False, use_tc_tiling_on_sc=True
    ),
    out_type=jax.ShapeDtypeStruct((_NUNIQUE, _BATCH), jnp.float32),
    scratch_types=[pltpu.VMEM((n, _CHUNK), jnp.float32) for n in _BANDS]
    + [pltpu.VMEM((_COLS_PER_W,), jnp.int32)]
    + [pltpu.SemaphoreType.DMA for _ in _BANDS],
)
def _onehot_sc(x_hbm, out_hbm, *refs):
    bufs = refs[:_NB]
    idx_v = refs[_NB]
    sems = refs[_NB + 1:]

    wid = lax.axis_index("s") * 2 + lax.axis_index("c")
    base_col = wid * _COLS_PER_W

    idx_copy = pltpu.async_copy(
        x_hbm.at[pl.ds(base_col * 1, _COLS_PER_W)], idx_v, sems[0]
    )

    zeros = jnp.zeros((_L,), jnp.float32)
    ones = jnp.ones((_L,), jnp.float32)
    lane = lax.iota(jnp.int32, _L)

    def zero_band(b):
        buf = bufs[b]

        def zero_body(i, carry):
            r = i * 8
            for u in range(8):
                for q in range(_CHUNK // _L):
                    buf[r + u, pl.ds(q * _L, _L)] = zeros
            return carry

        lax.fori_loop(0, _BANDS[b] // 8, zero_body, 0)

    def scatter(c, b, val):
        lo, hi = _LOS[b], _LOS[b] + _BANDS[b]
        for j in range(_CHUNK // _L):
            xv = idx_v[pl.ds(c * _CHUNK + j * _L, _L)]
            col = lane + j * _L
            mask = (xv >= lo) & (xv < hi) if lo > 0 else xv < hi
            plsc.store_scatter(bufs[b], [xv - lo, col], val, mask=mask)

    def dst(c, b):
        return out_hbm.at[
            pl.ds(_LOS[b], _BANDS[b]), pl.ds(base_col + c * _CHUNK, _CHUNK)
        ]

    zero_band(0)
    idx_copy.wait()
    scatter(0, 0, ones)
    pltpu.async_copy(bufs[0], dst(0, 0), sems[0])
    for b in range(1, _NB):
        zero_band(b)
        scatter(0, b, ones)
        pltpu.async_copy(bufs[b], dst(0, b), sems[b])

    for c in range(1, _NCHUNK):
        for b in range(_NB):
            pltpu.make_async_copy(bufs[b], dst(c - 1, b), sems[b]).wait()
            scatter(c - 1, b, zeros)
            scatter(c, b, ones)
            pltpu.async_copy(bufs[b], dst(c, b), sems[b])

    for b in range(_NB):
        pltpu.make_async_copy(bufs[b], dst(_NCHUNK - 1, b), sems[b]).wait()


def kernel(x):
    return _onehot_sc(x.astype(jnp.int32)).T


# speedup vs baseline: 1.0032x; 1.0032x over previous
import functools

import jax
import jax.numpy as jnp
from jax import lax
from jax.experimental import pallas as pl
from jax.experimental.pallas import tpu as pltpu
from jax.experimental.pallas import tpu_sc as plsc

_NUNIQUE = 1000
_BATCH = 16384
_NW = 32
_COLS_PER_W = _BATCH // _NW
_CHUNK = 128
_NCHUNK = _COLS_PER_W // _CHUNK
_L = 16
_H0 = 504
_H1 = _NUNIQUE - _H0

_mesh = plsc.VectorSubcoreMesh(core_axis_name="c", subcore_axis_name="s")


@functools.partial(
    pl.kernel,
    mesh=_mesh,
    compiler_params=pltpu.CompilerParams(
        needs_layout_passes=False, use_tc_tiling_on_sc=True
    ),
    out_type=jax.ShapeDtypeStruct((_NUNIQUE, _BATCH), jnp.float32),
    scratch_types=[
        pltpu.VMEM((_H0, _CHUNK), jnp.float32),
        pltpu.VMEM((_H1, _CHUNK), jnp.float32),
        pltpu.VMEM((_COLS_PER_W,), jnp.int32),
        pltpu.SemaphoreType.DMA,
        pltpu.SemaphoreType.DMA,
    ],
)
def _onehot_sc(x_hbm, out_hbm, buf0, buf1, idx_v, sem0, sem1):
    wid = lax.axis_index("s") * 2 + lax.axis_index("c")
    base_col = wid * _COLS_PER_W

    idx_copy = pltpu.async_copy(
        x_hbm.at[pl.ds(base_col * 1, _COLS_PER_W)], idx_v, sem0
    )

    zeros = jnp.zeros((_L,), jnp.float32)
    ones = jnp.ones((_L,), jnp.float32)
    lane = lax.iota(jnp.int32, _L)

    def zero_half(buf, nrows):
        def zero_body(i, carry):
            r = i * 8
            for u in range(8):
                for q in range(_CHUNK // _L):
                    buf[r + u, pl.ds(q * _L, _L)] = zeros
            return carry

        lax.fori_loop(0, nrows // 8, zero_body, 0)

    def scatter(c, half, val):
        buf, lo, n = (buf0, 0, _H0) if half == 0 else (buf1, _H0, _H1)
        for j in range(_CHUNK // _L):
            xv = idx_v[pl.ds(c * _CHUNK + j * _L, _L)]
            col = lane + j * _L
            if half == 0:
                mask = xv < _H0
                plsc.store_scatter(buf, [xv, col], val, mask=mask)
            else:
                mask = xv >= _H0
                plsc.store_scatter(buf, [xv - _H0, col], val, mask=mask)

    def dst(c, half):
        lo, n = (0, _H0) if half == 0 else (_H0, _H1)
        return out_hbm.at[pl.ds(lo, n), pl.ds(base_col + c * _CHUNK, _CHUNK)]

    bufs = (buf0, buf1)
    sems = (sem0, sem1)

    zero_half(buf0, _H0)
    idx_copy.wait()
    scatter(0, 0, ones)
    pltpu.async_copy(buf0, dst(0, 0), sem0)
    zero_half(buf1, _H1)
    scatter(0, 1, ones)
    pltpu.async_copy(buf1, dst(0, 1), sem1)

    for c in range(1, _NCHUNK):
        for half in (0, 1):
            pltpu.make_async_copy(bufs[half], dst(c - 1, half), sems[half]).wait()
            scatter(c - 1, half, zeros)
            scatter(c, half, ones)
            pltpu.async_copy(bufs[half], dst(c, half), sems[half])

    for half in (0, 1):
        pltpu.make_async_copy(
            bufs[half], dst(_NCHUNK - 1, half), sems[half]
        ).wait()


def kernel(x):
    return _onehot_sc(x.astype(jnp.int32)).T
